# baseline (speedup 1.0000x reference)
import functools

import jax
import jax.numpy as jnp
from jax import lax
from jax.experimental import pallas as pl
from jax.experimental.pallas import tpu as pltpu
from jax.experimental.pallas import tpu_sc as plsc

_VOCAB = 100000
_EMBED = 32
_B = 4096
_L = 200
_N = _B * _L

_NC = 2
_NS = 16
_NW = _NC * _NS
_PER_W = _N // _NW
_CHUNK = 1600
_NCHUNK = _PER_W // _CHUNK
_NPAIR = _NCHUNK // 2


def _emb_body(table_hbm, idx_hbm, out_hbm,
              idx_v0, idx_v1, rows_v0, rows_v1, sem0, sem1):
    wid = lax.axis_index("s") * _NC + lax.axis_index("c")
    base = wid * _PER_W
    bufs = ((idx_v0, rows_v0, sem0), (idx_v1, rows_v1, sem1))

    def load_and_fire(i, b):
        idx_v, rows_v, sem = bufs[b]
        pltpu.sync_copy(idx_hbm.at[pl.ds(base + i * _CHUNK, _CHUNK)], idx_v)
        pltpu.async_copy(table_hbm.at[idx_v], rows_v, sem)

    load_and_fire(0, 0)

    def pair(j, _):
        for b in range(2):
            i = 2 * j + b
            idx_v, rows_v, sem = bufs[b]
            if b == 0:
                load_and_fire(i + 1, 1)
            else:
                @pl.when(j < _NPAIR - 1)
                def _():
                    load_and_fire(i + 1, 0)
            pltpu.make_async_copy(table_hbm.at[idx_v], rows_v, sem).wait()
            pltpu.sync_copy(rows_v, out_hbm.at[pl.ds(base + i * _CHUNK, _CHUNK)])
        return 0

    lax.fori_loop(0, _NPAIR, pair, 0)


_emb = functools.partial(
    pl.kernel,
    mesh=plsc.VectorSubcoreMesh(core_axis_name="c", subcore_axis_name="s"),
    out_type=jax.ShapeDtypeStruct((_N, _EMBED), jnp.float32),
    scratch_types=[
        pltpu.VMEM((_CHUNK,), jnp.int32),
        pltpu.VMEM((_CHUNK,), jnp.int32),
        pltpu.VMEM((_CHUNK, _EMBED), jnp.float32),
        pltpu.VMEM((_CHUNK, _EMBED), jnp.float32),
        pltpu.SemaphoreType.DMA,
        pltpu.SemaphoreType.DMA,
    ],
    compiler_params=pltpu.CompilerParams(use_tc_tiling_on_sc=False),
)(_emb_body)


def kernel(indices, table):
    flat_idx = indices.reshape(_N).astype(jnp.int32)
    out = _emb(table, flat_idx)
    return out.reshape(_B, _L, _EMBED)


# speedup vs baseline: 1.8966x; 1.8966x over previous
import functools

import jax
import jax.numpy as jnp
from jax import lax
from jax.experimental import pallas as pl
from jax.experimental.pallas import tpu as pltpu
from jax.experimental.pallas import tpu_sc as plsc

_VOCAB = 100000
_EMBED = 32
_B = 4096
_L = 200

_NC = 2
_NS = 16
_NW = _NC * _NS
_BPW = _B // _NW
_BBLK = 8
_NBLK = _BPW // _BBLK
_NPAIR = _NBLK // 2


def _emb_body(table_hbm, idx_hbm, out_hbm,
              idx_v0, idx_v1, rows_v0, rows_v1, sem0, sem1):
    wid = lax.axis_index("s") * _NC + lax.axis_index("c")
    base = wid * _BPW
    bufs = ((idx_v0, rows_v0, sem0), (idx_v1, rows_v1, sem1))

    def load_and_fire(i, b):
        idx_v, rows_v, sem = bufs[b]
        pltpu.sync_copy(idx_hbm.at[pl.ds(base + i * _BBLK, _BBLK)], idx_v)
        for j in range(_BBLK):
            pltpu.async_copy(table_hbm.at[idx_v.at[j]], rows_v.at[j], sem)

    def drain_and_store(i, b):
        idx_v, rows_v, sem = bufs[b]
        for j in range(_BBLK):
            pltpu.make_async_copy(
                table_hbm.at[idx_v.at[j]], rows_v.at[j], sem).wait()
        pltpu.sync_copy(
            rows_v,
            out_hbm.at[pl.ds(base + i * _BBLK, _BBLK), slice(None),
                       pl.ds(0, _EMBED)])

    load_and_fire(0, 0)

    def pair(j, _):
        for b in range(2):
            i = 2 * j + b
            if b == 0:
                load_and_fire(i + 1, 1)
            else:
                @pl.when(j < _NPAIR - 1)
                def _():
                    load_and_fire(i + 1, 0)
            drain_and_store(i, b)
        return 0

    lax.fori_loop(0, _NPAIR, pair, 0)


_emb = functools.partial(
    pl.kernel,
    mesh=plsc.VectorSubcoreMesh(core_axis_name="c", subcore_axis_name="s"),
    out_type=jax.ShapeDtypeStruct((_B, _L, 128), jnp.float32),
    scratch_types=[
        pltpu.VMEM((_BBLK, _L), jnp.int32),
        pltpu.VMEM((_BBLK, _L), jnp.int32),
        pltpu.VMEM((_BBLK, _L, _EMBED), jnp.float32),
        pltpu.VMEM((_BBLK, _L, _EMBED), jnp.float32),
        pltpu.SemaphoreType.DMA,
        pltpu.SemaphoreType.DMA,
    ],
    compiler_params=pltpu.CompilerParams(use_tc_tiling_on_sc=False),
)(_emb_body)


def kernel(indices, table):
    return _emb(table, indices)[:, :, :_EMBED]
